# speedup vs baseline: 1.2588x; 1.2588x over previous
import functools
import math

import jax
import jax.numpy as jnp
from jax import lax
from jax.experimental import pallas as pl
from jax.experimental.pallas import tpu as pltpu
from jax.experimental.pallas import tpu_sc as plsc

_N = 100000
_D = 128
_DK = 32
_H = 2
_DV = _D // _H
_DFF = 256

_NW = 32
_CHUNK = 128
_BLK = 1000


def _sc_gather(x, idx):
    n4 = idx.shape[0]
    d = x.shape[1]
    n_chunks = n4 // _CHUNK
    per_w = (n_chunks + _NW - 1) // _NW
    mesh = plsc.VectorSubcoreMesh(core_axis_name="c", subcore_axis_name="s")

    @functools.partial(
        pl.kernel,
        out_type=jax.ShapeDtypeStruct((n4, d), jnp.float32),
        mesh=mesh,
        scratch_types=[
            pltpu.VMEM((_CHUNK,), jnp.int32),
            pltpu.VMEM((_CHUNK, d), jnp.float32),
            pltpu.SemaphoreType.DMA,
        ],
    )
    def gather_kernel(x_hbm, idx_hbm, out_hbm, idx_v, rows_v, sem):
        wid = lax.axis_index("s") * 2 + lax.axis_index("c")

        def body(i, carry):
            c = wid + i * _NW

            @pl.when(c < n_chunks)
            def _():
                base = pl.multiple_of(c * _CHUNK, _CHUNK)
                pltpu.sync_copy(idx_hbm.at[pl.ds(base, _CHUNK)], idx_v)
                pltpu.async_copy(x_hbm.at[idx_v], rows_v, sem).wait()
                pltpu.sync_copy(rows_v, out_hbm.at[pl.ds(base, _CHUNK)])

            return carry

        lax.fori_loop(0, per_w, body, 0)

    return gather_kernel(x, idx)


def _logits_block(xb, nbh_ref, wq_ref, wk_ref):
    inv = 1.0 / math.sqrt(_DK)
    cols = []
    for h in range(_H):
        q = jnp.dot(xb, wq_ref[h], preferred_element_type=jnp.float32)
        for r in range(5):
            src = xb if r == 0 else nbh_ref[r - 1]
            k = jnp.dot(src, wk_ref[h, r], preferred_element_type=jnp.float32)
            cols.append(jnp.sum(q * k, axis=1, keepdims=True) * inv)
    pad = jnp.full((xb.shape[0], 16 - 5 * _H), -1e30, jnp.float32)
    return jnp.concatenate(cols + [pad], axis=1)


def _stage1_kernel(x_ref, nbh_ref, wq_ref, wk_ref, ms_ref):
    i = pl.program_id(0)
    s_blk = _logits_block(x_ref[...], nbh_ref, wq_ref, wk_ref)

    @pl.when(i == 0)
    def _():
        ms_ref[0:1, :] = jnp.full((1, 16), -1e30, jnp.float32)
        ms_ref[1:2, :] = jnp.zeros((1, 16), jnp.float32)

    m_old = ms_ref[0:1, :]
    s_old = ms_ref[1:2, :]
    m_blk = jnp.max(s_blk, axis=0, keepdims=True)
    m_new = jnp.maximum(m_old, m_blk)
    e_blk = jnp.sum(jnp.exp(s_blk - m_new), axis=0, keepdims=True)
    ms_ref[0:1, :] = m_new
    ms_ref[1:2, :] = s_old * jnp.exp(m_old - m_new) + e_blk


def _layernorm(v, g_ref, b_ref):
    mu = jnp.mean(v, axis=1, keepdims=True)
    var = jnp.mean((v - mu) ** 2, axis=1, keepdims=True)
    return (v - mu) * jax.lax.rsqrt(var + 1e-5) * g_ref[...] + b_ref[...]


def _stage2_kernel(x_ref, nbh_ref, ms_ref, wq_ref, wk_ref, wv_ref,
                   w1_ref, b1_ref, w2_ref, b2_ref,
                   g1_ref, be1_ref, g2_ref, be2_ref, out_ref):
    xb = x_ref[...]
    s_blk = _logits_block(xb, nbh_ref, wq_ref, wk_ref)
    a_blk = jnp.exp(s_blk - ms_ref[0:1, :]) / ms_ref[1:2, :]

    z_parts = []
    for h in range(_H):
        zh = jnp.zeros((xb.shape[0], _DV), jnp.float32)
        for r in range(5):
            src = xb if r == 0 else nbh_ref[r - 1]
            v = jnp.dot(src, wv_ref[h, r], preferred_element_type=jnp.float32)
            zh = zh + a_blk[:, h * 5 + r:h * 5 + r + 1] * v
        z_parts.append(zh)
    z = jnp.concatenate(z_parts, axis=1)

    h1 = _layernorm(xb + z, g1_ref, be1_ref)
    ff = jnp.dot(jnp.maximum(
        jnp.dot(h1, w1_ref[...], preferred_element_type=jnp.float32)
        + b1_ref[...], 0.0), w2_ref[...], preferred_element_type=jnp.float32)
    ff = ff + b2_ref[...]
    out_ref[...] = _layernorm(h1 + ff, g2_ref, be2_ref)


def _full_spec(shape):
    return pl.BlockSpec(shape, lambda *_: tuple(0 for _ in shape))


def kernel(x, adjacency_matrix, w_q, w_k, w_v, W1, b1, W2, b2,
           g1, be1, g2, be2):
    n, d = x.shape
    grid = (n // _BLK,)

    idx = adjacency_matrix.astype(jnp.int32).T.reshape(-1)
    nbh = _sc_gather(x, idx).reshape(4, n, d)

    x_spec = pl.BlockSpec((_BLK, d), lambda i: (i, 0))
    nbh_spec = pl.BlockSpec((4, _BLK, d), lambda i: (0, i, 0))
    wq_spec = _full_spec(w_q.shape)
    wk_spec = _full_spec(w_k.shape)

    ms = pl.pallas_call(
        _stage1_kernel,
        grid=grid,
        in_specs=[x_spec, nbh_spec, wq_spec, wk_spec],
        out_specs=pl.BlockSpec((2, 16), lambda i: (0, 0)),
        out_shape=jax.ShapeDtypeStruct((2, 16), jnp.float32),
        compiler_params=pltpu.CompilerParams(
            dimension_semantics=("arbitrary",)),
    )(x, nbh, w_q, w_k)

    row = lambda v: v.reshape(1, -1)
    out = pl.pallas_call(
        _stage2_kernel,
        grid=grid,
        in_specs=[x_spec, nbh_spec, _full_spec((2, 16)), wq_spec, wk_spec,
                  _full_spec(w_v.shape), _full_spec(W1.shape),
                  _full_spec((1, _DFF)), _full_spec(W2.shape),
                  _full_spec((1, d)), _full_spec((1, d)), _full_spec((1, d)),
                  _full_spec((1, d)), _full_spec((1, d))],
        out_specs=x_spec,
        out_shape=jax.ShapeDtypeStruct((n, d), jnp.float32),
        compiler_params=pltpu.CompilerParams(
            dimension_semantics=("arbitrary",)),
    )(x, nbh, ms, w_q, w_k, w_v, W1, row(b1), W2, row(b2),
      row(g1), row(be1), row(g2), row(be2))
    return out


# speedup vs baseline: 1.6230x; 1.2893x over previous
import functools
import math

import jax
import jax.numpy as jnp
import numpy as np
from jax import lax
from jax.experimental import pallas as pl
from jax.experimental.pallas import tpu as pltpu
from jax.experimental.pallas import tpu_sc as plsc

_N = 100000
_D = 128
_DK = 32
_H = 2
_DV = _D // _H
_DFF = 256

_NW = 32
_CHUNK = 128
_BLK = 1000


def _sc_gather(x, idx):
    n4 = idx.shape[0]
    d = x.shape[1]
    n_chunks = n4 // _CHUNK
    per_w = (n_chunks + _NW - 1) // _NW
    mesh = plsc.VectorSubcoreMesh(core_axis_name="c", subcore_axis_name="s")

    @functools.partial(
        pl.kernel,
        out_type=jax.ShapeDtypeStruct((n4, d), jnp.float32),
        mesh=mesh,
        scratch_types=[
            pltpu.VMEM((_CHUNK,), jnp.int32),
            pltpu.VMEM((_CHUNK, d), jnp.float32),
            pltpu.SemaphoreType.DMA,
        ],
    )
    def gather_kernel(x_hbm, idx_hbm, out_hbm, idx_v, rows_v, sem):
        wid = lax.axis_index("s") * 2 + lax.axis_index("c")

        def body(i, carry):
            c = wid + i * _NW

            @pl.when(c < n_chunks)
            def _():
                base = pl.multiple_of(c * _CHUNK, _CHUNK)
                pltpu.sync_copy(idx_hbm.at[pl.ds(base, _CHUNK)], idx_v)
                pltpu.async_copy(x_hbm.at[idx_v], rows_v, sem).wait()
                pltpu.sync_copy(rows_v, out_hbm.at[pl.ds(base, _CHUNK)])

            return carry

        lax.fori_loop(0, per_w, body, 0)

    return gather_kernel(x, idx)


def _dot(a, b):
    return jnp.dot(a, b, preferred_element_type=jnp.float32)


def _logits_block(xb, nbh_ref, wqc_ref, wkc_ref, rmat_ref):
    qcat = _dot(xb, wqc_ref[...])
    s = None
    for r in range(5):
        src = xb if r == 0 else nbh_ref[r - 1]
        kcat = _dot(src, wkc_ref[r])
        part = _dot(qcat * kcat, rmat_ref[r])
        s = part if s is None else s + part
    return s


def _stage1_kernel(x_ref, nbh_ref, wqc_ref, wkc_ref, rmat_ref, ms_ref):
    i = pl.program_id(0)
    s_blk = _logits_block(x_ref[...], nbh_ref, wqc_ref, wkc_ref, rmat_ref)

    @pl.when(i == 0)
    def _():
        ms_ref[0:1, :] = jnp.full((1, 16), -1e30, jnp.float32)
        ms_ref[1:2, :] = jnp.zeros((1, 16), jnp.float32)

    m_old = ms_ref[0:1, :]
    s_old = ms_ref[1:2, :]
    m_blk = jnp.max(s_blk, axis=0, keepdims=True)
    m_new = jnp.maximum(m_old, m_blk)
    e_blk = jnp.sum(jnp.exp(s_blk - m_new), axis=0, keepdims=True)
    ms_ref[0:1, :] = m_new
    ms_ref[1:2, :] = s_old * jnp.exp(m_old - m_new) + e_blk


def _layernorm(v, g_ref, b_ref):
    mu = jnp.mean(v, axis=1, keepdims=True)
    var = jnp.mean((v - mu) ** 2, axis=1, keepdims=True)
    return (v - mu) * jax.lax.rsqrt(var + 1e-5) * g_ref[...] + b_ref[...]


def _stage2_kernel(x_ref, nbh_ref, ms_ref, wqc_ref, wkc_ref, rmat_ref,
                   sel_ref, wvc_ref, w1_ref, b1_ref, w2_ref, b2_ref,
                   g1_ref, be1_ref, g2_ref, be2_ref, out_ref):
    xb = x_ref[...]
    s_blk = _logits_block(xb, nbh_ref, wqc_ref, wkc_ref, rmat_ref)
    a_blk = jnp.exp(s_blk - ms_ref[0:1, :]) / ms_ref[1:2, :]

    z = None
    for r in range(5):
        src = xb if r == 0 else nbh_ref[r - 1]
        term = _dot(a_blk, sel_ref[r]) * _dot(src, wvc_ref[r])
        z = term if z is None else z + term

    h1 = _layernorm(xb + z, g1_ref, be1_ref)
    ff = _dot(jnp.maximum(_dot(h1, w1_ref[...]) + b1_ref[...], 0.0),
              w2_ref[...]) + b2_ref[...]
    out_ref[...] = _layernorm(h1 + ff, g2_ref, be2_ref)


def _full_spec(shape):
    return pl.BlockSpec(shape, lambda *_: tuple(0 for _ in shape))


def _selection_mats():
    rmat = np.zeros((5, _H * _DK, 16), np.float32)
    sel = np.zeros((5, 16, _D), np.float32)
    inv = 1.0 / math.sqrt(_DK)
    for r in range(5):
        for h in range(_H):
            rmat[r, h * _DK:(h + 1) * _DK, h * 5 + r] = inv
            sel[r, h * 5 + r, h * _DV:(h + 1) * _DV] = 1.0
    return jnp.asarray(rmat), jnp.asarray(sel)


def kernel(x, adjacency_matrix, w_q, w_k, w_v, W1, b1, W2, b2,
           g1, be1, g2, be2):
    n, d = x.shape
    grid = (n // _BLK,)

    idx = adjacency_matrix.astype(jnp.int32).T.reshape(-1)
    nbh = _sc_gather(x, idx).reshape(4, n, d)

    wqc = jnp.concatenate([w_q[h] for h in range(_H)], axis=1)
    wkc = jnp.stack([jnp.concatenate([w_k[h, r] for h in range(_H)], axis=1)
                     for r in range(5)])
    wvc = jnp.stack([jnp.concatenate([w_v[h, r] for h in range(_H)], axis=1)
                     for r in range(5)])
    rmat, sel = _selection_mats()

    x_spec = pl.BlockSpec((_BLK, d), lambda i: (i, 0))
    nbh_spec = pl.BlockSpec((4, _BLK, d), lambda i: (0, i, 0))

    ms = pl.pallas_call(
        _stage1_kernel,
        grid=grid,
        in_specs=[x_spec, nbh_spec, _full_spec(wqc.shape),
                  _full_spec(wkc.shape), _full_spec(rmat.shape)],
        out_specs=pl.BlockSpec((2, 16), lambda i: (0, 0)),
        out_shape=jax.ShapeDtypeStruct((2, 16), jnp.float32),
        compiler_params=pltpu.CompilerParams(
            dimension_semantics=("arbitrary",)),
    )(x, nbh, wqc, wkc, rmat)

    row = lambda v: v.reshape(1, -1)
    out = pl.pallas_call(
        _stage2_kernel,
        grid=grid,
        in_specs=[x_spec, nbh_spec, _full_spec((2, 16)),
                  _full_spec(wqc.shape), _full_spec(wkc.shape),
                  _full_spec(rmat.shape), _full_spec(sel.shape),
                  _full_spec(wvc.shape), _full_spec(W1.shape),
                  _full_spec((1, _DFF)), _full_spec(W2.shape),
                  _full_spec((1, d)), _full_spec((1, d)), _full_spec((1, d)),
                  _full_spec((1, d)), _full_spec((1, d))],
        out_specs=x_spec,
        out_shape=jax.ShapeDtypeStruct((n, d), jnp.float32),
        compiler_params=pltpu.CompilerParams(
            dimension_semantics=("arbitrary",)),
    )(x, nbh, ms, wqc, wkc, rmat, sel, wvc, W1, row(b1), W2, row(b2),
      row(g1), row(be1), row(g2), row(be2))
    return out
